# baseline (speedup 1.0000x reference)
import functools

import jax
import jax.numpy as jnp
from jax import lax
from jax.experimental import pallas as pl
from jax.experimental.pallas import tpu as pltpu
from jax.experimental.pallas import tpu_sc as plsc

_NC = 2
_NS = 16
_NW = _NC * _NS
_L = 16

_T = 1000


def _mesh():
    return plsc.VectorSubcoreMesh(core_axis_name="c", subcore_axis_name="s")


_SC_PARAMS = pltpu.CompilerParams(use_tc_tiling_on_sc=False)


def _store_heads(r, nh, out_refs):
    if nh == 4:
        ha_ref, hb_ref, as_ref, ad_ref = out_refs
        ha_ref[0] = r[:, 0:16]
        ha_ref[1] = r[:, 16:32]
        hb_ref[0] = r[:, 32:48]
        hb_ref[1] = r[:, 48:64]
    else:
        ha_ref, as_ref, ad_ref = out_refs
        ha_ref[...] = r[:, 0:16]
    as_ref[...] = r[:, 16 * nh:16 * nh + 16]
    ad_ref[...] = r[:, 16 * nh + 16:16 * nh + 32]


def _node_outs(n, nh):
    if nh == 4:
        return (
            [
                pl.BlockSpec((2, 2000, 16), lambda i: (0, i, 0)),
                pl.BlockSpec((2, 2000, 16), lambda i: (0, i, 0)),
                pl.BlockSpec((2000, 16), lambda i: (i, 0)),
                pl.BlockSpec((2000, 16), lambda i: (i, 0)),
            ],
            [
                jax.ShapeDtypeStruct((2, n, 16), jnp.float32),
                jax.ShapeDtypeStruct((2, n, 16), jnp.float32),
                jax.ShapeDtypeStruct((n, 16), jnp.float32),
                jax.ShapeDtypeStruct((n, 16), jnp.float32),
            ],
        )
    return (
        [pl.BlockSpec((2000, 16), lambda i: (i, 0))] * 3,
        [jax.ShapeDtypeStruct((n, 16), jnp.float32)] * 3,
    )


def _node1_body(x_ref, w_ref, *out_refs):
    x = x_ref[...]
    r = x[:, 0:1] * w_ref[0:1, :] + x[:, 1:2] * w_ref[1:2, :]
    _store_heads(r, 4, out_refs)


def _node1(x, wcat):
    n = x.shape[0]
    BN = 2000
    out_specs, out_shape = _node_outs(n, 4)
    return pl.pallas_call(
        _node1_body,
        grid=(n // BN,),
        in_specs=[
            pl.BlockSpec((BN, 2), lambda i: (i, 0)),
            pl.BlockSpec((2, 96), lambda i: (0, 0)),
        ],
        out_specs=out_specs,
        out_shape=out_shape,
    )(x, wcat)


def _assemble(qa, qb, den, b):
    d = den[0] + den[1] + 1e-16
    cols = [
        qa[0] / d[:, 0:1],
        qa[1] / d[:, 1:2],
        qb[0] / d[:, 2:3],
        qb[1] / d[:, 3:4],
    ]
    xx = jnp.concatenate(cols, axis=1)
    return jnp.maximum(xx + b, 0.0)


def _node23_body(qa_ref, qb_ref, den_ref, b_ref, w_ref, *out_refs):
    xx = _assemble(qa_ref[...], qb_ref[...], den_ref[...], b_ref[...])
    r = jnp.dot(xx, w_ref[...], preferred_element_type=jnp.float32)
    nh = (w_ref.shape[1] - 32) // 16
    _store_heads(r, nh, out_refs)


def _node23(qa, qb, den, b, wcat, nh):
    n = qa.shape[1]
    BN = 2000
    wcols = 16 * nh + 32
    out_specs, out_shape = _node_outs(n, nh)
    return pl.pallas_call(
        _node23_body,
        grid=(n // BN,),
        in_specs=[
            pl.BlockSpec((2, BN, 16), lambda i: (0, i, 0)),
            pl.BlockSpec((2, BN, 16), lambda i: (0, i, 0)),
            pl.BlockSpec((2, BN, 16), lambda i: (0, i, 0)),
            pl.BlockSpec((1, 64), lambda i: (0, 0)),
            pl.BlockSpec((64, wcols), lambda i: (0, 0)),
        ],
        out_specs=out_specs,
        out_shape=out_shape,
    )(qa, qb, den, b.reshape(1, 64), wcat)


def _edge_body(ea_ref, u_ref, o1_ref, o2_ref, o3_ref):
    ea = ea_ref[...]
    r = ea[:, 0:1] * u_ref[0:1, :] + ea[:, 1:2] * u_ref[1:2, :]
    o1_ref[...] = r[:, 0:16]
    o2_ref[...] = r[:, 16:32]
    o3_ref[...] = r[:, 32:48]


def _edge_ae(ea, ucat):
    e = ea.shape[0]
    BE = 8000
    outs = [jax.ShapeDtypeStruct((e, 16), jnp.float32)] * 3
    return pl.pallas_call(
        _edge_body,
        grid=(e // BE,),
        in_specs=[
            pl.BlockSpec((BE, 2), lambda i: (i, 0)),
            pl.BlockSpec((2, 48), lambda i: (0, 0)),
        ],
        out_specs=[pl.BlockSpec((BE, 16), lambda i: (i, 0))] * 3,
        out_shape=outs,
    )(ea, ucat)


def _fc_body(p3_ref, den_ref, b3_ref, w_ref, bfc_ref, o_ref):
    acc = p3_ref[0] + p3_ref[1]
    d = den_ref[0, :, 0:1] + den_ref[1, :, 0:1] + 1e-16
    xx = jnp.maximum(acc / d + b3_ref[...], 0.0)
    o_ref[...] = jnp.dot(xx, w_ref[...],
                         preferred_element_type=jnp.float32) + bfc_ref[...]


def _fc(p3, den3, b3, wfc, bfc):
    n = p3.shape[1]
    BN = 2000
    out_dim = wfc.shape[1]
    return pl.pallas_call(
        _fc_body,
        grid=(n // BN,),
        in_specs=[
            pl.BlockSpec((2, BN, 16), lambda i: (0, i, 0)),
            pl.BlockSpec((2, BN, 16), lambda i: (0, i, 0)),
            pl.BlockSpec((1, 16), lambda i: (0, 0)),
            pl.BlockSpec((16, out_dim), lambda i: (0, 0)),
            pl.BlockSpec((1, out_dim), lambda i: (0, 0)),
        ],
        out_specs=pl.BlockSpec((BN, out_dim), lambda i: (i, 0)),
        out_shape=jax.ShapeDtypeStruct((n, out_dim), jnp.float32),
    )(p3, den3, b3.reshape(1, 16), wfc, bfc.reshape(1, out_dim))


def _write_back(sh_ref, hbm_ref, c, s, n):
    rows = (n // _NS + 7) // 8 * 8
    last = n - (_NS - 1) * rows

    @pl.when(s < _NS - 1)
    def _():
        pltpu.sync_copy(sh_ref.at[pl.ds(s * rows, rows)],
                        hbm_ref.at[c, pl.ds(s * rows, rows)])

    @pl.when(s == _NS - 1)
    def _():
        pltpu.sync_copy(sh_ref.at[pl.ds((_NS - 1) * rows, last)],
                        hbm_ref.at[c, pl.ds((_NS - 1) * rows, last)])


def _zero_sh(z_hbm, sh_ref, s, n):
    rows = (n // _NS + 7) // 8 * 8
    last = n - (_NS - 1) * rows

    @pl.when(s < _NS - 1)
    def _():
        pltpu.sync_copy(z_hbm.at[pl.ds(s * rows, rows)],
                        sh_ref.at[pl.ds(s * rows, rows)])

    @pl.when(s == _NS - 1)
    def _():
        pltpu.sync_copy(z_hbm.at[pl.ds((_NS - 1) * rows, last)],
                        sh_ref.at[pl.ds((_NS - 1) * rows, last)])


def _layer_call(src, dst, asrc, adst, ae, ha, hb, zeros16):
    n = asrc.shape[0]
    e = src.shape[0]
    ew1 = e // _NW
    nt1 = ew1 // _T
    ew2 = e // _NS
    nt2 = ew2 // _T

    def body(src_hbm, dst_hbm, asrc_hbm, adst_hbm, ae_hbm, ha_hbm, hb_hbm,
             z_hbm, ex_hbm, den_hbm, qa_hbm, qb_hbm,
             srcv, dstv, g1v, g2v, aev, exv, acc_sh):
        c = lax.axis_index("c")
        s = lax.axis_index("s")

        _zero_sh(z_hbm, acc_sh, s, n)
        plsc.subcore_barrier()

        base = (c * _NS + s) * ew1

        @pl.loop(0, nt1)
        def _(t):
            b = base + t * _T
            pltpu.sync_copy(src_hbm.at[pl.ds(b, _T)], srcv)
            pltpu.sync_copy(dst_hbm.at[pl.ds(b, _T)], dstv)
            pltpu.sync_copy(ae_hbm.at[pl.ds(b, _T)], aev)
            pltpu.sync_copy(asrc_hbm.at[srcv], g1v)
            pltpu.sync_copy(adst_hbm.at[dstv], g2v)

            @plsc.parallel_loop(0, _T, unroll=8)
            def _(i):
                a = g1v[i] + g2v[i] + aev[i]
                a = jnp.maximum(a, a * 0.2)
                exv[i] = jnp.exp(a)

            pltpu.sync_copy(exv, ex_hbm.at[pl.ds(b, _T)])
            pltpu.sync_copy(exv, acc_sh.at[dstv], add=True)

        plsc.subcore_barrier()
        _write_back(acc_sh, den_hbm, c, s, n)

        for htab_hbm, q_hbm, head0 in ((ha_hbm, qa_hbm, 0), (hb_hbm, qb_hbm, 2)):
            _zero_sh(z_hbm, acc_sh, s, n)
            plsc.subcore_barrier()

            base2 = s * ew2

            @pl.loop(0, nt2)
            def _(t):
                b = base2 + t * _T
                pltpu.sync_copy(src_hbm.at[pl.ds(b, _T)], srcv)
                pltpu.sync_copy(dst_hbm.at[pl.ds(b, _T)], dstv)
                pltpu.sync_copy(ex_hbm.at[pl.ds(b, _T)], exv)
                pltpu.sync_copy(htab_hbm.at[c].at[srcv], g1v)

                @plsc.parallel_loop(0, _T, unroll=8)
                def _(t2):
                    exw = exv[t2]
                    m = jnp.where(c == 0, exw[head0], exw[head0 + 1])
                    g1v[t2] = g1v[t2] * m

                pltpu.sync_copy(g1v, acc_sh.at[dstv], add=True)

            plsc.subcore_barrier()
            _write_back(acc_sh, q_hbm, c, s, n)
            plsc.subcore_barrier()

    f = pl.kernel(
        body,
        out_type=[
            jax.ShapeDtypeStruct((e, 16), jnp.float32),
            jax.ShapeDtypeStruct((_NC, n, 16), jnp.float32),
            jax.ShapeDtypeStruct((_NC, n, 16), jnp.float32),
            jax.ShapeDtypeStruct((_NC, n, 16), jnp.float32),
        ],
        mesh=_mesh(),
        scratch_types=[
            pltpu.VMEM((_T,), jnp.int32),
            pltpu.VMEM((_T,), jnp.int32),
            pltpu.VMEM((_T, _L), jnp.float32),
            pltpu.VMEM((_T, _L), jnp.float32),
            pltpu.VMEM((_T, _L), jnp.float32),
            pltpu.VMEM((_T, _L), jnp.float32),
            pltpu.VMEM_SHARED((n, _L), jnp.float32),
        ],
        compiler_params=_SC_PARAMS,
    )
    return f(src, dst, asrc, adst, ae, ha, hb, zeros16)


def _layer3_call(src, dst, asrc, adst, ae, htab, zeros16):
    n = asrc.shape[0]
    e = src.shape[0]
    ew = e // _NW
    nt = ew // _T

    def body(src_hbm, dst_hbm, asrc_hbm, adst_hbm, ae_hbm, h_hbm, z_hbm,
             ex_hbm, den_hbm, q_hbm,
             srcv, dstv, g1v, g2v, aev, exv, acc_sh):
        c = lax.axis_index("c")
        s = lax.axis_index("s")

        _zero_sh(z_hbm, acc_sh, s, n)
        plsc.subcore_barrier()

        base = (c * _NS + s) * ew

        @pl.loop(0, nt)
        def _(t):
            b = base + t * _T
            pltpu.sync_copy(src_hbm.at[pl.ds(b, _T)], srcv)
            pltpu.sync_copy(dst_hbm.at[pl.ds(b, _T)], dstv)
            pltpu.sync_copy(ae_hbm.at[pl.ds(b, _T)], aev)
            pltpu.sync_copy(asrc_hbm.at[srcv], g1v)
            pltpu.sync_copy(adst_hbm.at[dstv], g2v)

            @plsc.parallel_loop(0, _T, unroll=8)
            def _(i):
                a = g1v[i] + g2v[i] + aev[i]
                a = jnp.maximum(a, a * 0.2)
                exv[i] = jnp.exp(a)

            pltpu.sync_copy(exv, ex_hbm.at[pl.ds(b, _T)])
            pltpu.sync_copy(exv, acc_sh.at[dstv], add=True)

        plsc.subcore_barrier()
        _write_back(acc_sh, den_hbm, c, s, n)
        _zero_sh(z_hbm, acc_sh, s, n)
        plsc.subcore_barrier()

        @pl.loop(0, nt)
        def _(t):
            b = base + t * _T
            pltpu.sync_copy(src_hbm.at[pl.ds(b, _T)], srcv)
            pltpu.sync_copy(dst_hbm.at[pl.ds(b, _T)], dstv)
            pltpu.sync_copy(ex_hbm.at[pl.ds(b, _T)], exv)
            pltpu.sync_copy(h_hbm.at[srcv], g1v)

            @plsc.parallel_loop(0, _T, unroll=8)
            def _(t2):
                exw = exv[t2]
                g1v[t2] = g1v[t2] * exw[0]

            pltpu.sync_copy(g1v, acc_sh.at[dstv], add=True)

        plsc.subcore_barrier()
        _write_back(acc_sh, q_hbm, c, s, n)

    f = pl.kernel(
        body,
        out_type=[
            jax.ShapeDtypeStruct((e, 16), jnp.float32),
            jax.ShapeDtypeStruct((_NC, n, 16), jnp.float32),
            jax.ShapeDtypeStruct((_NC, n, 16), jnp.float32),
        ],
        mesh=_mesh(),
        scratch_types=[
            pltpu.VMEM((_T,), jnp.int32),
            pltpu.VMEM((_T,), jnp.int32),
            pltpu.VMEM((_T, _L), jnp.float32),
            pltpu.VMEM((_T, _L), jnp.float32),
            pltpu.VMEM((_T, _L), jnp.float32),
            pltpu.VMEM((_T, _L), jnp.float32),
            pltpu.VMEM_SHARED((n, _L), jnp.float32),
        ],
        compiler_params=_SC_PARAMS,
    )
    return f(src, dst, asrc, adst, ae, htab, zeros16)


def _fold(W, a_src, a_dst):
    heads = a_src.shape[1]
    ch = a_src.shape[2]
    Wr = W.reshape(W.shape[0], heads, ch)
    Us = jnp.einsum('khc,hc->kh', Wr, a_src[0])
    Ud = jnp.einsum('khc,hc->kh', Wr, a_dst[0])
    return Us, Ud


def _pad16(u):
    k, h = u.shape
    return jnp.concatenate([u, jnp.zeros((k, 16 - h), u.dtype)], axis=1)


def kernel(x, edge_index, edge_attr,
           W1, a_src1, a_dst1, a_e1, We1, b1,
           W2, a_src2, a_dst2, a_e2, We2, b2,
           W3, a_src3, a_dst3, a_e3, We3, b3,
           Wfc, bfc):
    n = x.shape[0]
    src = edge_index[0].astype(jnp.int32)
    dst = edge_index[1].astype(jnp.int32)

    Us1, Ud1 = _fold(W1, a_src1, a_dst1)
    Us2, Ud2 = _fold(W2, a_src2, a_dst2)
    Us3, Ud3 = _fold(W3, a_src3, a_dst3)
    Ue1 = jnp.einsum('khc,hc->kh', We1.reshape(2, 4, 16), a_e1[0])
    Ue2 = jnp.einsum('khc,hc->kh', We2.reshape(2, 4, 16), a_e2[0])
    Ue3 = jnp.einsum('khc,hc->kh', We3.reshape(2, 1, 16), a_e3[0])

    wcat1 = jnp.concatenate([W1, _pad16(Us1), _pad16(Ud1)], axis=1)
    wcat2 = jnp.concatenate([W2, _pad16(Us2), _pad16(Ud2)], axis=1)
    wcat3 = jnp.concatenate([W3, _pad16(Us3), _pad16(Ud3)], axis=1)
    uecat = jnp.concatenate([_pad16(Ue1), _pad16(Ue2), _pad16(Ue3)],
                            axis=1)

    zeros16 = jnp.zeros((n, 16), jnp.float32)

    ae1, ae2, ae3 = _edge_ae(edge_attr, uecat)

    ha, hb, asrc, adst = _node1(x, wcat1)
    _, den1, qa, qb = _layer_call(src, dst, asrc, adst, ae1, ha, hb, zeros16)

    ha, hb, asrc, adst = _node23(qa, qb, den1, b1, wcat2, 4)
    _, den2, qa, qb = _layer_call(src, dst, asrc, adst, ae2, ha, hb, zeros16)

    hh, asrc, adst = _node23(qa, qb, den2, b2, wcat3, 1)
    _, den3, p3 = _layer3_call(src, dst, asrc, adst, ae3, hh, zeros16)

    return _fc(p3, den3, b3, Wfc, bfc)
